# baseline (speedup 1.0000x reference)
import functools
import jax
import jax.numpy as jnp
from jax import lax
from jax.experimental import pallas as pl
from jax.experimental.pallas import tpu as pltpu
from jax.experimental.pallas import tpu_sc as plsc

_NC = 2
_NS = 16
_NW = _NC * _NS
_LANES = 16


def _make_sc_add(B, L, D):
    ROWS = 8
    l_per_w = L // _NW
    n_sub = l_per_w // ROWS
    GROUPS = D // _LANES
    RING = 3

    mesh = plsc.VectorSubcoreMesh(core_axis_name="c", subcore_axis_name="s")

    @functools.partial(
        pl.kernel,
        mesh=mesh,
        out_type=jax.ShapeDtypeStruct((B, L, D), jnp.float32),
        scratch_types=(
            [pltpu.VMEM((ROWS, D), jnp.float32) for _ in range(RING * B)]
            + [pltpu.VMEM((ROWS, D), jnp.float32) for _ in range(2)]
            + [pltpu.SemaphoreType.DMA for _ in range(RING)]
            + [pltpu.SemaphoreType.DMA for _ in range(RING)]
            + [pltpu.SemaphoreType.DMA for _ in range(2)]
        ),
    )
    def k(x_hbm, pe_hbm, o_hbm, *refs):
        xb = refs[0:RING * B]
        peb = refs[RING * B:RING * B + 2]
        lsem = refs[RING * B + 2:RING * B + 2 + RING]
        ssem = refs[RING * B + 2 + RING:RING * B + 2 + 2 * RING]
        psem = refs[RING * B + 2 + 2 * RING:]

        wid = lax.axis_index("s") * _NC + lax.axis_index("c")
        base_l = wid * l_per_w

        def l0(t):
            return base_l + t * ROWS

        def load_chunk(t):
            q = t % RING
            return [
                pltpu.async_copy(
                    x_hbm.at[b, pl.ds(l0(t), ROWS)], xb[q * B + b], lsem[q])
                for b in range(B)
            ]

        pe_pend = {}
        for t in range(min(2, n_sub)):
            pe_pend[t] = pltpu.async_copy(
                pe_hbm.at[pl.ds(l0(t), ROWS)], peb[t % 2], psem[t % 2])
        ld = {}
        for t in range(min(2, n_sub)):
            ld[t] = load_chunk(t)

        st = {}
        for t in range(n_sub):
            q = t % RING
            for h in ld[t]:
                h.wait()
            pe_pend[t].wait()
            pv = peb[t % 2]
            xset = [xb[q * B + b] for b in range(B)]

            def add_col(j, carry):
                col = pl.ds(j * _LANES, _LANES)
                for r in range(ROWS):
                    pvreg = pv[r, col]
                    for b in range(B):
                        xv = xset[b]
                        xv[r, col] = xv[r, col] + pvreg
                return carry

            lax.fori_loop(0, GROUPS, add_col, 0)

            st[t] = [
                pltpu.async_copy(
                    xset[b], o_hbm.at[b, pl.ds(l0(t), ROWS)], ssem[q])
                for b in range(B)
            ]

            if t + 2 < n_sub:
                pe_pend[t + 2] = pltpu.async_copy(
                    pe_hbm.at[pl.ds(l0(t + 2), ROWS)], peb[t % 2], psem[t % 2])
                if t >= 1:
                    for h in st[t - 1]:
                        h.wait()
                ld[t + 2] = load_chunk(t + 2)

        for t in range(max(0, n_sub - 3), n_sub):
            for h in st[t]:
                h.wait()

    return k


def _tc_add_block(x_ref, pe_ref, o_ref):
    o_ref[...] = x_ref[...] + pe_ref[...]


def _tc_add(x, pe):
    B, L, D = x.shape
    LB = 512
    grid = (L // LB, B)
    return pl.pallas_call(
        _tc_add_block,
        grid=grid,
        in_specs=[
            pl.BlockSpec((1, LB, D), lambda j, b: (b, j, 0)),
            pl.BlockSpec((LB, D), lambda j, b: (j, 0)),
        ],
        out_specs=pl.BlockSpec((1, LB, D), lambda j, b: (b, j, 0)),
        out_shape=jax.ShapeDtypeStruct((B, L, D), x.dtype),
    )(x, pe)


def kernel(x, pe):
    B, L, D = x.shape
    pe = pe[:L]
    B_SC = 1
    out_sc = _make_sc_add(B_SC, L, D)(x[B - B_SC:], pe)
    out_tc = _tc_add(x[:B - B_SC], pe)
    return jnp.concatenate([out_tc, out_sc], axis=0)


# speedup vs baseline: 2.0006x; 2.0006x over previous
import functools
import jax
import jax.numpy as jnp
from jax import lax
from jax.experimental import pallas as pl
from jax.experimental.pallas import tpu as pltpu
from jax.experimental.pallas import tpu_sc as plsc

_NC = 2
_NS = 16
_NW = _NC * _NS
_LANES = 16


def _make_sc_add(B, L, D):
    ROWS = 8
    l_per_w = L // _NW
    n_sub = l_per_w // ROWS
    GROUPS = D // _LANES
    RING = 3

    mesh = plsc.VectorSubcoreMesh(core_axis_name="c", subcore_axis_name="s")

    @functools.partial(
        pl.kernel,
        mesh=mesh,
        out_type=jax.ShapeDtypeStruct((B, L, D), jnp.float32),
        scratch_types=(
            [pltpu.VMEM((ROWS, D), jnp.float32) for _ in range(RING * B)]
            + [pltpu.VMEM((ROWS, D), jnp.float32) for _ in range(2)]
            + [pltpu.SemaphoreType.DMA for _ in range(RING)]
            + [pltpu.SemaphoreType.DMA for _ in range(RING)]
            + [pltpu.SemaphoreType.DMA for _ in range(2)]
        ),
    )
    def k(x_hbm, pe_hbm, o_hbm, *refs):
        xb = refs[0:RING * B]
        peb = refs[RING * B:RING * B + 2]
        lsem = refs[RING * B + 2:RING * B + 2 + RING]
        ssem = refs[RING * B + 2 + RING:RING * B + 2 + 2 * RING]
        psem = refs[RING * B + 2 + 2 * RING:]

        wid = lax.axis_index("s") * _NC + lax.axis_index("c")
        base_l = wid * l_per_w

        def l0(t):
            return base_l + t * ROWS

        def load_chunk(t):
            q = t % RING
            return [
                pltpu.async_copy(
                    x_hbm.at[b, pl.ds(l0(t), ROWS)], xb[q * B + b], lsem[q])
                for b in range(B)
            ]

        pe_pend = {}
        for t in range(min(2, n_sub)):
            pe_pend[t] = pltpu.async_copy(
                pe_hbm.at[pl.ds(l0(t), ROWS)], peb[t % 2], psem[t % 2])
        ld = {}
        for t in range(min(2, n_sub)):
            ld[t] = load_chunk(t)

        st = {}
        for t in range(n_sub):
            q = t % RING
            for h in ld[t]:
                h.wait()
            pe_pend[t].wait()
            pv = peb[t % 2]
            xset = [xb[q * B + b] for b in range(B)]

            def add_col(j, carry):
                col = pl.ds(j * _LANES, _LANES)
                for r in range(ROWS):
                    pvreg = pv[r, col]
                    for b in range(B):
                        xv = xset[b]
                        xv[r, col] = xv[r, col] + pvreg
                return carry

            lax.fori_loop(0, GROUPS, add_col, 0)

            st[t] = [
                pltpu.async_copy(
                    xset[b], o_hbm.at[b, pl.ds(l0(t), ROWS)], ssem[q])
                for b in range(B)
            ]

            if t + 2 < n_sub:
                pe_pend[t + 2] = pltpu.async_copy(
                    pe_hbm.at[pl.ds(l0(t + 2), ROWS)], peb[t % 2], psem[t % 2])
                if t >= 1:
                    for h in st[t - 1]:
                        h.wait()
                ld[t + 2] = load_chunk(t + 2)

        for t in range(max(0, n_sub - 3), n_sub):
            for h in st[t]:
                h.wait()

    return k


def kernel(x, pe):
    B, L, D = x.shape
    return _make_sc_add(B, L, D)(x, pe[:L])


# speedup vs baseline: 2.0025x; 1.0009x over previous
import functools
import jax
import jax.numpy as jnp
from jax import lax
from jax.experimental import pallas as pl
from jax.experimental.pallas import tpu as pltpu
from jax.experimental.pallas import tpu_sc as plsc

_NC = 2
_NS = 16
_NW = _NC * _NS
_LANES = 16


def _make_sc_add(B, L, D):
    ROWS = 8
    l_per_w = L // _NW
    n_sub = l_per_w // ROWS
    GROUPS = D // _LANES
    RING = 3

    mesh = plsc.VectorSubcoreMesh(core_axis_name="c", subcore_axis_name="s")

    @functools.partial(
        pl.kernel,
        mesh=mesh,
        out_type=jax.ShapeDtypeStruct((B, L, D), jnp.float32),
        scratch_types=(
            [pltpu.VMEM((B, ROWS, D), jnp.float32) for _ in range(RING)]
            + [pltpu.VMEM((ROWS, D), jnp.float32) for _ in range(2)]
            + [pltpu.SemaphoreType.DMA for _ in range(RING)]
            + [pltpu.SemaphoreType.DMA for _ in range(RING)]
            + [pltpu.SemaphoreType.DMA for _ in range(2)]
        ),
    )
    def k(x_hbm, pe_hbm, o_hbm, *refs):
        xb = refs[0:RING]
        peb = refs[RING:RING + 2]
        lsem = refs[RING + 2:RING + 2 + RING]
        ssem = refs[RING + 2 + RING:RING + 2 + 2 * RING]
        psem = refs[RING + 2 + 2 * RING:]

        wid = lax.axis_index("s") * _NC + lax.axis_index("c")
        base_l = wid * l_per_w

        def l0(t):
            return base_l + t * ROWS

        def load_chunk(t):
            q = t % RING
            return [pltpu.async_copy(
                x_hbm.at[:, pl.ds(l0(t), ROWS)], xb[q], lsem[q])]

        pe_pend = {}
        for t in range(min(2, n_sub)):
            pe_pend[t] = pltpu.async_copy(
                pe_hbm.at[pl.ds(l0(t), ROWS)], peb[t % 2], psem[t % 2])
        ld = {}
        for t in range(min(2, n_sub)):
            ld[t] = load_chunk(t)

        st = {}
        for t in range(n_sub):
            q = t % RING
            for h in ld[t]:
                h.wait()
            pe_pend[t].wait()
            pv = peb[t % 2]
            xv = xb[q]

            def add_col(j, carry):
                col = pl.ds(j * _LANES, _LANES)
                for r in range(ROWS):
                    pvreg = pv[r, col]
                    for b in range(B):
                        xv[b, r, col] = xv[b, r, col] + pvreg
                return carry

            lax.fori_loop(0, GROUPS, add_col, 0)

            st[t] = [pltpu.async_copy(
                xv, o_hbm.at[:, pl.ds(l0(t), ROWS)], ssem[q])]

            if t + 2 < n_sub:
                pe_pend[t + 2] = pltpu.async_copy(
                    pe_hbm.at[pl.ds(l0(t + 2), ROWS)], peb[t % 2], psem[t % 2])
                if t >= 1:
                    for h in st[t - 1]:
                        h.wait()
                ld[t + 2] = load_chunk(t + 2)

        for t in range(max(0, n_sub - 3), n_sub):
            for h in st[t]:
                h.wait()

    return k


def kernel(x, pe):
    B, L, D = x.shape
    return _make_sc_add(B, L, D)(x, pe[:L])
